# baseline (speedup 1.0000x reference)
import functools

import jax
import jax.numpy as jnp
import numpy as np
from jax import lax
from jax.experimental import pallas as pl
from jax.experimental.pallas import tpu as pltpu
from jax.experimental.pallas import tpu_sc as plsc

B, T, V, P = 1024, 6, 100, 20
NC, NS = 2, 16
NW = NC * NS
BPW = B // NW
VC = 7
BIG = np.float32(1e30)
MASKED = np.float32(1e6)

_EGO_W = BPW * T * 2
_SCORE_W = BPW * V * 3
_LANE_B = V * P * 2
_OUT_C = 8
_OUT_W = BPW * _OUT_C


def _xform_x(v, ok):
    return jnp.where(ok, v * 30.0 - 15.0, MASKED)


def _xform_y(v, ok):
    return jnp.where(ok, v * 60.0 - 30.0, MASKED)


def _sc_body(ego_hbm, lanes_hbm, scores_hbm, out_hbm, ego_v, scores_v,
             lane_v, out_v):
    cid = lax.axis_index("c")
    sid = lax.axis_index("s")
    wid = sid * NC + cid
    base_b = wid * BPW
    iota = lax.iota(jnp.int32, 16)
    fzero = jnp.zeros((16,), jnp.float32)

    pltpu.sync_copy(ego_hbm.at[pl.ds(wid * _EGO_W, _EGO_W)], ego_v)
    pltpu.sync_copy(scores_hbm.at[pl.ds(wid * _SCORE_W, _SCORE_W)], scores_v)

    rowoff = iota * (P * 2)

    def per_b(lb, _):
        pltpu.sync_copy(
            lanes_hbm.at[pl.ds((base_b + lb) * _LANE_B, _LANE_B)], lane_v)

        exi = jnp.minimum(lb * (T * 2) + 2 * iota, _EGO_W - 2)
        exs = jnp.where(iota < T, plsc.load_gather(ego_v, [exi]), fzero)
        eys = jnp.where(iota < T, plsc.load_gather(ego_v, [exi + 1]), fzero)
        cumx = plsc.cumsum(exs)
        cumy = plsc.cumsum(eys)
        pxs = [jnp.sum(jnp.where(iota == t, cumx, fzero)) for t in range(T)]
        pys = [jnp.sum(jnp.where(iota == t, cumy, fzero)) for t in range(T)]
        px_spl = [jnp.broadcast_to(pxs[t], (16,)) for t in range(T)]
        py_spl = [jnp.broadcast_to(pys[t], (16,)) for t in range(T)]

        best2 = [BIG] * T
        bestv = [np.int32(0)] * T
        gmin2 = [BIG] * T
        for c in range(VC):
            vvec = iota + c * 16
            sci = jnp.minimum(lb * (V * 3) + c * 48 + 3 * iota + 2,
                              _SCORE_W - 1)
            svec = plsc.load_gather(scores_v, [sci])
            okc = (svec >= 0.5) & (vvec < V)

            def dist_p(p, minps):
                xi = jnp.minimum(rowoff + (c * 640 + 2 * p), _LANE_B - 2)
                xg = plsc.load_gather(lane_v, [xi])
                yg = plsc.load_gather(lane_v, [xi + 1])
                x = _xform_x(xg, okc)
                y = _xform_y(yg, okc)
                out = []
                for t in range(T):
                    dx = x - px_spl[t]
                    dy = y - py_spl[t]
                    out.append(jnp.minimum(minps[t], dx * dx + dy * dy))
                return tuple(out)

            minp = lax.fori_loop(0, P, dist_p, tuple([jnp.full((16,), BIG)] * T))

            valid = vvec < V
            for t in range(T):
                m = jnp.min(minp[t])
                vidx = jnp.min(jnp.where((minp[t] == m) & valid, vvec, 1000))
                upd = m < best2[t]
                best2[t] = jnp.where(upd, m, best2[t])
                bestv[t] = jnp.where(upd, vidx, bestv[t])
                gmin2[t] = jnp.minimum(gmin2[t], m)

        outrow = jnp.full((16,), BIG)
        hit_sofar = np.bool_(False)
        for t in range(T):
            vb = bestv[t]
            rowbase = vb * (P * 2)
            sci = jnp.minimum(lb * (V * 3) + 3 * vb + 2, _SCORE_W - 1)
            okv = plsc.load_gather(
                scores_v, [jnp.broadcast_to(sci, (16,))]) >= 0.5
            sx_s = px_spl[t - 1] if t > 0 else fzero
            sy_s = py_spl[t - 1] if t > 0 else fzero
            d1x = px_spl[t] - sx_s
            d1y = py_spl[t] - sy_s
            hit_t = np.bool_(False)
            for (p0, nseg) in ((0, 16), (16, P - 1 - 16)):
                si = jnp.minimum(rowbase + 2 * (p0 + iota), _LANE_B - 2)
                ei = jnp.minimum(rowbase + 2 * (p0 + iota) + 2, _LANE_B - 2)
                ax = _xform_x(plsc.load_gather(lane_v, [si]), okv)
                ay = _xform_y(plsc.load_gather(lane_v, [si + 1]), okv)
                bx = _xform_x(plsc.load_gather(lane_v, [ei]), okv)
                by = _xform_y(plsc.load_gather(lane_v, [ei + 1]), okv)
                d2x = bx - ax
                d2y = by - ay
                cross = d1x * d2y - d1y * d2x
                tsx = ax - sx_s
                tsy = ay - sy_s
                t1 = (tsx * d2y - tsy * d2x) / cross
                t2 = (tsx * d1y - tsy * d1x) / cross
                hitv = ((t1 >= 0.0) & (t1 <= 1.0) & (t2 >= 0.0) & (t2 <= 1.0)
                        & (iota < nseg))
                hit_t = hit_t | jnp.any(hitv)
            hit_sofar = hit_sofar | hit_t
            val = jnp.where(hit_sofar, BIG, gmin2[t])
            outrow = jnp.where(iota == t, jnp.broadcast_to(val, (16,)), outrow)

        plsc.store_scatter(out_v, [jnp.minimum(lb * _OUT_C + iota, _OUT_W - 1)],
                           outrow, mask=iota < _OUT_C)
        return 0

    lax.fori_loop(0, BPW, per_b, 0)
    pltpu.sync_copy(out_v, out_hbm.at[pl.ds(wid * _OUT_W, _OUT_W)])


@functools.partial(jax.jit)
def _sc_call(ego, lanes, scores):
    mesh = plsc.VectorSubcoreMesh(core_axis_name="c", subcore_axis_name="s")
    f = functools.partial(
        pl.kernel,
        mesh=mesh,
        compiler_params=pltpu.CompilerParams(needs_layout_passes=False),
        out_type=jax.ShapeDtypeStruct((B * _OUT_C,), jnp.float32),
        scratch_types=[
            pltpu.VMEM((_EGO_W,), jnp.float32),
            pltpu.VMEM((_SCORE_W,), jnp.float32),
            pltpu.VMEM((_LANE_B,), jnp.float32),
            pltpu.VMEM((_OUT_W,), jnp.float32),
        ],
    )(_sc_body)
    return f(ego, lanes, scores)


def kernel(ego_fut_preds, lane_preds, lane_score_preds):
    ego = ego_fut_preds.reshape(-1)
    lanes = lane_preds.reshape(-1)
    scores = lane_score_preds.reshape(-1)
    out = _sc_call(ego, lanes, scores).reshape(B, _OUT_C)[:, :T]
    d = jnp.sqrt(out)
    loss = jnp.maximum(np.float32(0.0), np.float32(1.0) - d)
    return jnp.mean(loss)


# speedup vs baseline: 5.0833x; 5.0833x over previous
import functools

import jax
import jax.numpy as jnp
import numpy as np
from jax import lax
from jax.experimental import pallas as pl
from jax.experimental.pallas import tpu as pltpu
from jax.experimental.pallas import tpu_sc as plsc

B, T, V, P = 1024, 6, 100, 20
NC, NS, L = 2, 16, 16
NW = NC * NS
NCHUNK = B // L
CPW = NCHUNK // NW
BIG = np.float32(1e30)
MASKED = np.float32(1e6)

_LANE_C = V * P * 2 * L
_SC_C = V * L
_EGO_C = T * 2 * L
_OUT_C = 8 * L


def _sc_body(ego_hbm, lanes_hbm, scores_hbm, out_hbm, lane_v, sc_v, ego_v,
             out_v):
    cid = lax.axis_index("c")
    sid = lax.axis_index("s")
    wid = sid * NC + cid
    iota = lax.iota(jnp.int32, L)

    def per_chunk(k, _):
        chunk = wid * CPW + k
        pltpu.sync_copy(lanes_hbm.at[pl.ds(chunk * _LANE_C, _LANE_C)], lane_v)
        pltpu.sync_copy(scores_hbm.at[pl.ds(chunk * _SC_C, _SC_C)], sc_v)
        pltpu.sync_copy(ego_hbm.at[pl.ds(chunk * _EGO_C, _EGO_C)], ego_v)

        px, py = [], []
        runx = jnp.zeros((L,), jnp.float32)
        runy = jnp.zeros((L,), jnp.float32)
        for t in range(T):
            runx = runx + ego_v[pl.ds((2 * t) * L, L)]
            runy = runy + ego_v[pl.ds((2 * t + 1) * L, L)]
            px.append(runx)
            py.append(runy)
        dmask = []
        for t in range(T):
            dx = MASKED - px[t]
            dy = MASKED - py[t]
            dmask.append(dx * dx + dy * dy)

        def vbody(v, carry):
            best2, bestv = carry
            ok = sc_v[pl.ds(v * L, L)] >= 0.5
            base = v * (P * 2 * L)
            minp = [jnp.full((L,), BIG)] * T
            for p in range(P):
                x = lane_v[pl.ds(base + (2 * p) * L, L)] * 30.0 - 15.0
                y = lane_v[pl.ds(base + (2 * p + 1) * L, L)] * 60.0 - 30.0
                for t in range(T):
                    dx = x - px[t]
                    dy = y - py[t]
                    minp[t] = jnp.minimum(minp[t], dx * dx + dy * dy)
            new_best2, new_bestv = [], []
            for t in range(T):
                mval = jnp.where(ok, minp[t], dmask[t])
                upd = mval < best2[t]
                new_best2.append(jnp.where(upd, mval, best2[t]))
                new_bestv.append(jnp.where(upd, v, bestv[t]))
            return (tuple(new_best2), tuple(new_bestv))

        best2, bestv = lax.fori_loop(
            0, V, vbody,
            (tuple([jnp.full((L,), BIG)] * T),
             tuple([jnp.zeros((L,), jnp.int32)] * T)))

        hit_sofar = jnp.zeros((L,), jnp.bool_)
        for t in range(T):
            rowb = bestv[t] * (P * 2 * L) + iota
            okb = plsc.load_gather(sc_v, [bestv[t] * L + iota]) >= 0.5
            sxs = px[t - 1] if t > 0 else jnp.zeros((L,), jnp.float32)
            sys_ = py[t - 1] if t > 0 else jnp.zeros((L,), jnp.float32)
            d1x = px[t] - sxs
            d1y = py[t] - sys_
            hit_t = jnp.zeros((L,), jnp.bool_)
            prev_x = prev_y = None
            for p in range(P):
                gx = plsc.load_gather(lane_v, [rowb + (2 * p) * L])
                gy = plsc.load_gather(lane_v, [rowb + (2 * p + 1) * L])
                cur_x = jnp.where(okb, gx * 30.0 - 15.0, MASKED)
                cur_y = jnp.where(okb, gy * 60.0 - 30.0, MASKED)
                if p > 0:
                    d2x = cur_x - prev_x
                    d2y = cur_y - prev_y
                    cross = d1x * d2y - d1y * d2x
                    tsx = prev_x - sxs
                    tsy = prev_y - sys_
                    t1 = (tsx * d2y - tsy * d2x) / cross
                    t2 = (tsx * d1y - tsy * d1x) / cross
                    hit_t = hit_t | ((t1 >= 0.0) & (t1 <= 1.0)
                                     & (t2 >= 0.0) & (t2 <= 1.0))
                prev_x, prev_y = cur_x, cur_y
            hit_sofar = hit_sofar | hit_t
            out_v[pl.ds(t * L, L)] = jnp.where(hit_sofar, BIG, best2[t])
        out_v[pl.ds(6 * L, L)] = jnp.full((L,), BIG)
        out_v[pl.ds(7 * L, L)] = jnp.full((L,), BIG)
        pltpu.sync_copy(out_v, out_hbm.at[pl.ds(chunk * _OUT_C, _OUT_C)])
        return 0

    lax.fori_loop(0, CPW, per_chunk, 0)


@functools.partial(jax.jit)
def _sc_call(ego, lanes, scores):
    mesh = plsc.VectorSubcoreMesh(core_axis_name="c", subcore_axis_name="s")
    f = functools.partial(
        pl.kernel,
        mesh=mesh,
        compiler_params=pltpu.CompilerParams(needs_layout_passes=False),
        out_type=jax.ShapeDtypeStruct((NCHUNK * _OUT_C,), jnp.float32),
        scratch_types=[
            pltpu.VMEM((_LANE_C,), jnp.float32),
            pltpu.VMEM((_SC_C,), jnp.float32),
            pltpu.VMEM((_EGO_C,), jnp.float32),
            pltpu.VMEM((_OUT_C,), jnp.float32),
        ],
    )(_sc_body)
    return f(ego, lanes, scores)


def kernel(ego_fut_preds, lane_preds, lane_score_preds):
    lanes = (lane_preds.transpose(1, 2, 3, 0)
             .reshape(V * P * 2, NCHUNK, L).swapaxes(0, 1).reshape(-1))
    ego = (ego_fut_preds.transpose(1, 2, 0)
           .reshape(T * 2, NCHUNK, L).swapaxes(0, 1).reshape(-1))
    scores = (lane_score_preds[:, :, 2].transpose(1, 0)
              .reshape(V, NCHUNK, L).swapaxes(0, 1).reshape(-1))
    out = _sc_call(ego, lanes, scores).reshape(NCHUNK, 8, L)[:, :T, :]
    d = jnp.sqrt(out)
    loss = jnp.maximum(np.float32(0.0), np.float32(1.0) - d)
    return jnp.mean(loss)


# speedup vs baseline: 14.5796x; 2.8681x over previous
import functools

import jax
import jax.numpy as jnp
import numpy as np
from jax import lax
from jax.experimental import pallas as pl
from jax.experimental.pallas import tpu as pltpu
from jax.experimental.pallas import tpu_sc as plsc

B, T, V, P = 1024, 6, 100, 20
NC, NS, L = 2, 16, 16
NW = NC * NS
NCHUNK = B // L
CPW = NCHUNK // NW
BIG = np.float32(1e30)
MASKED = np.float32(1e6)

_SC_C = V * L
_EGO_C = T * 2 * L
_OUT_C = 8 * L


def _sc_body(ego_hbm, lanes_hbm, scores_hbm, out_hbm, lane_v, sc_v, ego_v,
             out_v):
    cid = lax.axis_index("c")
    sid = lax.axis_index("s")
    wid = sid * NC + cid
    iota = lax.iota(jnp.int32, L)

    def per_chunk(k, _):
        chunk = wid * CPW + k
        q = chunk // 8
        m = chunk % 8
        pltpu.sync_copy(lanes_hbm.at[:, :, q, :, pl.ds(m * L, L)], lane_v)
        pltpu.sync_copy(scores_hbm.at[pl.ds(chunk * _SC_C, _SC_C)], sc_v)
        pltpu.sync_copy(ego_hbm.at[pl.ds(chunk * _EGO_C, _EGO_C)], ego_v)

        px, py = [], []
        runx = jnp.zeros((L,), jnp.float32)
        runy = jnp.zeros((L,), jnp.float32)
        for t in range(T):
            runx = runx + ego_v[pl.ds((2 * t) * L, L)]
            runy = runy + ego_v[pl.ds((2 * t + 1) * L, L)]
            px.append(runx)
            py.append(runy)
        dmask = []
        for t in range(T):
            dx = MASKED - px[t]
            dy = MASKED - py[t]
            dmask.append(dx * dx + dy * dy)

        best2_all = [None] * T
        bestv_all = [None] * T
        for ts in (range(0, 3), range(3, 6)):
            ts = list(ts)

            def vbody(v, carry, ts=ts):
                best2, bestv = carry
                ok = sc_v[pl.ds(v * L, L)] >= 0.5
                minp = [jnp.full((L,), BIG)] * len(ts)
                for p in range(P):
                    x = lane_v[v, p, 0, :] * 30.0 - 15.0
                    y = lane_v[v, p, 1, :] * 60.0 - 30.0
                    for i, t in enumerate(ts):
                        dx = x - px[t]
                        dy = y - py[t]
                        minp[i] = jnp.minimum(minp[i], dx * dx + dy * dy)
                new_best2, new_bestv = [], []
                for i, t in enumerate(ts):
                    mval = jnp.where(ok, minp[i], dmask[t])
                    upd = mval < best2[i]
                    new_best2.append(jnp.where(upd, mval, best2[i]))
                    new_bestv.append(jnp.where(upd, v, bestv[i]))
                return (tuple(new_best2), tuple(new_bestv))

            b2, bv = lax.fori_loop(
                0, V, vbody,
                (tuple([jnp.full((L,), BIG)] * len(ts)),
                 tuple([jnp.zeros((L,), jnp.int32)] * len(ts))))
            for i, t in enumerate(ts):
                best2_all[t] = b2[i]
                bestv_all[t] = bv[i]

        hit_sofar = jnp.zeros((L,), jnp.bool_)
        for t in range(T):
            bv = bestv_all[t]
            okb = plsc.load_gather(sc_v, [bv * L + iota]) >= 0.5
            sxs = px[t - 1] if t > 0 else jnp.zeros((L,), jnp.float32)
            sys_ = py[t - 1] if t > 0 else jnp.zeros((L,), jnp.float32)
            d1x = px[t] - sxs
            d1y = py[t] - sys_
            hit_t = jnp.zeros((L,), jnp.bool_)
            prev_x = prev_y = None
            for p in range(P):
                pf = jnp.full((L,), p, jnp.int32)
                gx = plsc.load_gather(
                    lane_v, [bv, pf, jnp.zeros((L,), jnp.int32), iota])
                gy = plsc.load_gather(
                    lane_v, [bv, pf, jnp.ones((L,), jnp.int32), iota])
                cur_x = jnp.where(okb, gx * 30.0 - 15.0, MASKED)
                cur_y = jnp.where(okb, gy * 60.0 - 30.0, MASKED)
                if p > 0:
                    d2x = cur_x - prev_x
                    d2y = cur_y - prev_y
                    cross = d1x * d2y - d1y * d2x
                    tsx = prev_x - sxs
                    tsy = prev_y - sys_
                    t1 = (tsx * d2y - tsy * d2x) / cross
                    t2 = (tsx * d1y - tsy * d1x) / cross
                    hit_t = hit_t | ((t1 >= 0.0) & (t1 <= 1.0)
                                     & (t2 >= 0.0) & (t2 <= 1.0))
                prev_x, prev_y = cur_x, cur_y
            hit_sofar = hit_sofar | hit_t
            out_v[pl.ds(t * L, L)] = jnp.where(hit_sofar, BIG, best2_all[t])
        out_v[pl.ds(6 * L, L)] = jnp.full((L,), BIG)
        out_v[pl.ds(7 * L, L)] = jnp.full((L,), BIG)
        pltpu.sync_copy(out_v, out_hbm.at[pl.ds(chunk * _OUT_C, _OUT_C)])
        return 0

    lax.fori_loop(0, CPW, per_chunk, 0)


@functools.partial(jax.jit)
def _sc_call(ego, lanes5, scores):
    mesh = plsc.VectorSubcoreMesh(core_axis_name="c", subcore_axis_name="s")
    f = functools.partial(
        pl.kernel,
        mesh=mesh,
        compiler_params=pltpu.CompilerParams(
            needs_layout_passes=False, use_tc_tiling_on_sc=False),
        out_type=jax.ShapeDtypeStruct((NCHUNK * _OUT_C,), jnp.float32),
        scratch_types=[
            pltpu.VMEM((V, P, 2, L), jnp.float32),
            pltpu.VMEM((_SC_C,), jnp.float32),
            pltpu.VMEM((_EGO_C,), jnp.float32),
            pltpu.VMEM((_OUT_C,), jnp.float32),
        ],
    )(_sc_body)
    return f(ego, lanes5, scores)


def kernel(ego_fut_preds, lane_preds, lane_score_preds):
    lanes5 = (lane_preds.reshape(8, 128, V, P, 2)
              .transpose(2, 3, 0, 4, 1))
    ego = (ego_fut_preds.transpose(1, 2, 0)
           .reshape(T * 2, NCHUNK, L).swapaxes(0, 1).reshape(-1))
    scores = (lane_score_preds[:, :, 2].transpose(1, 0)
              .reshape(V, NCHUNK, L).swapaxes(0, 1).reshape(-1))
    out = _sc_call(ego, lanes5, scores).reshape(NCHUNK, 8, L)[:, :T, :]
    d = jnp.sqrt(out)
    loss = jnp.maximum(np.float32(0.0), np.float32(1.0) - d)
    return jnp.mean(loss)


# speedup vs baseline: 28.7254x; 1.9702x over previous
import functools

import jax
import jax.numpy as jnp
import numpy as np
from jax import lax
from jax.experimental import pallas as pl
from jax.experimental.pallas import tpu as pltpu
from jax.experimental.pallas import tpu_sc as plsc

B, T, V, P = 1024, 6, 100, 20
NC, NS, L = 2, 16, 16
NW = NC * NS
NCHUNK = B // L
CPW = NCHUNK // NW
BIG = np.float32(1e30)
MASKED = np.float32(1e6)

_SC_C = V * L
_EGO_C = T * 2 * L
_OUT_C = 8 * L


def _sc_body(ego_hbm, lanes_hbm, scores_hbm, out_hbm, lane_v, sc_v, ego_v,
             out_v):
    cid = lax.axis_index("c")
    sid = lax.axis_index("s")
    wid = sid * NC + cid
    iota = lax.iota(jnp.int32, L)

    def per_chunk(k, _):
        chunk = wid * CPW + k
        q = chunk // 8
        m = chunk % 8
        pltpu.sync_copy(lanes_hbm.at[:, :, q, :, pl.ds(m * L, L)], lane_v)
        pltpu.sync_copy(scores_hbm.at[pl.ds(chunk * _SC_C, _SC_C)], sc_v)
        pltpu.sync_copy(ego_hbm.at[pl.ds(chunk * _EGO_C, _EGO_C)], ego_v)

        px, py = [], []
        runx = jnp.zeros((L,), jnp.float32)
        runy = jnp.zeros((L,), jnp.float32)
        for t in range(T):
            runx = runx + ego_v[pl.ds((2 * t) * L, L)]
            runy = runy + ego_v[pl.ds((2 * t + 1) * L, L)]
            px.append(runx)
            py.append(runy)
        dmask = []
        for t in range(T):
            dx = MASKED - px[t]
            dy = MASKED - py[t]
            dmask.append(dx * dx + dy * dy)

        best2_all = [None] * T
        bestv_all = [None] * T
        for ts in (range(0, 3), range(3, 6)):
            ts = list(ts)
            init = (tuple([jnp.full((L,), BIG)] * len(ts)),
                    tuple([jnp.zeros((L,), jnp.int32)] * len(ts)))

            def vbody(v, carry, ts=ts):
                best2, bestv = carry
                ok = sc_v[pl.ds(v * L, L)] >= 0.5
                minp0 = tuple([jnp.full((L,), BIG)] * len(ts))

                def pbody(p, minp, ts=ts, v=v):
                    x = lane_v[v, p, 0, :] * 30.0 - 15.0
                    y = lane_v[v, p, 1, :] * 60.0 - 30.0
                    out = []
                    for i, t in enumerate(ts):
                        dx = x - px[t]
                        dy = y - py[t]
                        out.append(
                            jnp.minimum(minp[i], dx * dx + dy * dy))
                    return tuple(out)

                minp = plsc.parallel_loop(0, P, unroll=4, carry=minp0)(pbody)
                new_best2, new_bestv = [], []
                for i, t in enumerate(ts):
                    mval = jnp.where(ok, minp[i], dmask[t])
                    upd = mval < best2[i]
                    new_best2.append(jnp.where(upd, mval, best2[i]))
                    new_bestv.append(jnp.where(upd, v, bestv[i]))
                return (tuple(new_best2), tuple(new_bestv))

            b2, bv = lax.fori_loop(0, V, vbody, init)
            for i, t in enumerate(ts):
                best2_all[t] = b2[i]
                bestv_all[t] = bv[i]

        hit_sofar = jnp.zeros((L,), jnp.bool_)
        for t in range(T):
            bv = bestv_all[t]
            okb = plsc.load_gather(sc_v, [bv * L + iota]) >= 0.5
            sxs = px[t - 1] if t > 0 else jnp.zeros((L,), jnp.float32)
            sys_ = py[t - 1] if t > 0 else jnp.zeros((L,), jnp.float32)
            d1x = px[t] - sxs
            d1y = py[t] - sys_
            hit_t = jnp.zeros((L,), jnp.bool_)
            prev_x = prev_y = None
            for p in range(P):
                pf = jnp.full((L,), p, jnp.int32)
                gx = plsc.load_gather(
                    lane_v, [bv, pf, jnp.zeros((L,), jnp.int32), iota])
                gy = plsc.load_gather(
                    lane_v, [bv, pf, jnp.ones((L,), jnp.int32), iota])
                cur_x = jnp.where(okb, gx * 30.0 - 15.0, MASKED)
                cur_y = jnp.where(okb, gy * 60.0 - 30.0, MASKED)
                if p > 0:
                    d2x = cur_x - prev_x
                    d2y = cur_y - prev_y
                    cross = d1x * d2y - d1y * d2x
                    tsx = prev_x - sxs
                    tsy = prev_y - sys_
                    t1 = (tsx * d2y - tsy * d2x) / cross
                    t2 = (tsx * d1y - tsy * d1x) / cross
                    hit_t = hit_t | ((t1 >= 0.0) & (t1 <= 1.0)
                                     & (t2 >= 0.0) & (t2 <= 1.0))
                prev_x, prev_y = cur_x, cur_y
            hit_sofar = hit_sofar | hit_t
            out_v[pl.ds(t * L, L)] = jnp.where(hit_sofar, BIG, best2_all[t])
        out_v[pl.ds(6 * L, L)] = jnp.full((L,), BIG)
        out_v[pl.ds(7 * L, L)] = jnp.full((L,), BIG)
        pltpu.sync_copy(out_v, out_hbm.at[pl.ds(chunk * _OUT_C, _OUT_C)])
        return 0

    lax.fori_loop(0, CPW, per_chunk, 0)


@functools.partial(jax.jit)
def _sc_call(ego, lanes5, scores):
    mesh = plsc.VectorSubcoreMesh(core_axis_name="c", subcore_axis_name="s")
    f = functools.partial(
        pl.kernel,
        mesh=mesh,
        compiler_params=pltpu.CompilerParams(
            needs_layout_passes=False, use_tc_tiling_on_sc=False),
        out_type=jax.ShapeDtypeStruct((NCHUNK * _OUT_C,), jnp.float32),
        scratch_types=[
            pltpu.VMEM((V, P, 2, L), jnp.float32),
            pltpu.VMEM((_SC_C,), jnp.float32),
            pltpu.VMEM((_EGO_C,), jnp.float32),
            pltpu.VMEM((_OUT_C,), jnp.float32),
        ],
    )(_sc_body)
    return f(ego, lanes5, scores)


def kernel(ego_fut_preds, lane_preds, lane_score_preds):
    lanes5 = (lane_preds.reshape(8, 128, V, P, 2)
              .transpose(2, 3, 0, 4, 1))
    ego = (ego_fut_preds.transpose(1, 2, 0)
           .reshape(T * 2, NCHUNK, L).swapaxes(0, 1).reshape(-1))
    scores = (lane_score_preds[:, :, 2].transpose(1, 0)
              .reshape(V, NCHUNK, L).swapaxes(0, 1).reshape(-1))
    out = _sc_call(ego, lanes5, scores).reshape(NCHUNK, 8, L)[:, :T, :]
    d = jnp.sqrt(out)
    loss = jnp.maximum(np.float32(0.0), np.float32(1.0) - d)
    return jnp.mean(loss)


# speedup vs baseline: 29.2906x; 1.0197x over previous
import functools

import jax
import jax.numpy as jnp
import numpy as np
from jax import lax
from jax.experimental import pallas as pl
from jax.experimental.pallas import tpu as pltpu
from jax.experimental.pallas import tpu_sc as plsc

B, T, V, P = 1024, 6, 100, 20
NC, NS, L = 2, 16, 16
NW = NC * NS
NCHUNK = B // L
CPW = NCHUNK // NW
BIG = np.float32(1e30)
MASKED = np.float32(1e6)

_SC_C = V * L
_EGO_C = T * 2 * L
_OUT_C = 8 * L


def _sc_body(ego_hbm, lanes_hbm, scores_hbm, out_hbm, lane_v0, lane_v1,
             sc_v, ego_v, out_v, sem0, sem1):
    cid = lax.axis_index("c")
    sid = lax.axis_index("s")
    wid = sid * NC + cid
    iota = lax.iota(jnp.int32, L)

    copies = []
    for k, (buf, sem) in enumerate(((lane_v0, sem0), (lane_v1, sem1))):
        chunk = wid * CPW + k
        q = chunk // 8
        m = chunk % 8
        copies.append(pltpu.async_copy(
            lanes_hbm.at[:, :, q, :, pl.ds(m * L, L)], buf, sem))

    for k, lane_v in enumerate((lane_v0, lane_v1)):
        chunk = wid * CPW + k
        q = chunk // 8
        m = chunk % 8
        pltpu.sync_copy(scores_hbm.at[pl.ds(chunk * _SC_C, _SC_C)], sc_v)
        pltpu.sync_copy(ego_hbm.at[:, q, :, pl.ds(m * L, L)], ego_v)
        copies[k].wait()

        px, py = [], []
        runx = jnp.zeros((L,), jnp.float32)
        runy = jnp.zeros((L,), jnp.float32)
        for t in range(T):
            runx = runx + ego_v[t, 0, :]
            runy = runy + ego_v[t, 1, :]
            px.append(runx)
            py.append(runy)
        dmask = []
        for t in range(T):
            dx = MASKED - px[t]
            dy = MASKED - py[t]
            dmask.append(dx * dx + dy * dy)

        best2_all = [None] * T
        bestv_all = [None] * T
        for ts in (range(0, 3), range(3, 6)):
            ts = list(ts)
            init = (tuple([jnp.full((L,), BIG)] * len(ts)),
                    tuple([jnp.zeros((L,), jnp.int32)] * len(ts)))

            def vbody(v, carry, ts=ts):
                best2, bestv = carry
                ok = sc_v[pl.ds(v * L, L)] >= 0.5
                minp0 = tuple([jnp.full((L,), BIG)] * len(ts))

                def pbody(p, minp, ts=ts, v=v):
                    x = lane_v[v, p, 0, :] * 30.0 - 15.0
                    y = lane_v[v, p, 1, :] * 60.0 - 30.0
                    out = []
                    for i, t in enumerate(ts):
                        dx = x - px[t]
                        dy = y - py[t]
                        out.append(
                            jnp.minimum(minp[i], dx * dx + dy * dy))
                    return tuple(out)

                minp = plsc.parallel_loop(0, P, unroll=4, carry=minp0)(pbody)
                new_best2, new_bestv = [], []
                for i, t in enumerate(ts):
                    mval = jnp.where(ok, minp[i], dmask[t])
                    upd = mval < best2[i]
                    new_best2.append(jnp.where(upd, mval, best2[i]))
                    new_bestv.append(jnp.where(upd, v, bestv[i]))
                return (tuple(new_best2), tuple(new_bestv))

            b2, bv = lax.fori_loop(0, V, vbody, init)
            for i, t in enumerate(ts):
                best2_all[t] = b2[i]
                bestv_all[t] = bv[i]

        hit_sofar = jnp.zeros((L,), jnp.bool_)
        for t in range(T):
            bv = bestv_all[t]
            okb = plsc.load_gather(sc_v, [bv * L + iota]) >= 0.5
            sxs = px[t - 1] if t > 0 else jnp.zeros((L,), jnp.float32)
            sys_ = py[t - 1] if t > 0 else jnp.zeros((L,), jnp.float32)
            d1x = px[t] - sxs
            d1y = py[t] - sys_
            hit_t = jnp.zeros((L,), jnp.bool_)
            prev_x = prev_y = None
            for p in range(P):
                pf = jnp.full((L,), p, jnp.int32)
                gx = plsc.load_gather(
                    lane_v, [bv, pf, jnp.zeros((L,), jnp.int32), iota])
                gy = plsc.load_gather(
                    lane_v, [bv, pf, jnp.ones((L,), jnp.int32), iota])
                cur_x = jnp.where(okb, gx * 30.0 - 15.0, MASKED)
                cur_y = jnp.where(okb, gy * 60.0 - 30.0, MASKED)
                if p > 0:
                    d2x = cur_x - prev_x
                    d2y = cur_y - prev_y
                    cross = d1x * d2y - d1y * d2x
                    tsx = prev_x - sxs
                    tsy = prev_y - sys_
                    t1 = (tsx * d2y - tsy * d2x) / cross
                    t2 = (tsx * d1y - tsy * d1x) / cross
                    hit_t = hit_t | ((t1 >= 0.0) & (t1 <= 1.0)
                                     & (t2 >= 0.0) & (t2 <= 1.0))
                prev_x, prev_y = cur_x, cur_y
            hit_sofar = hit_sofar | hit_t
            out_v[pl.ds(t * L, L)] = jnp.where(hit_sofar, BIG, best2_all[t])
        out_v[pl.ds(6 * L, L)] = jnp.full((L,), BIG)
        out_v[pl.ds(7 * L, L)] = jnp.full((L,), BIG)
        pltpu.sync_copy(out_v, out_hbm.at[pl.ds(chunk * _OUT_C, _OUT_C)])


@functools.partial(jax.jit)
def _sc_call(ego, lanes5, scores):
    mesh = plsc.VectorSubcoreMesh(core_axis_name="c", subcore_axis_name="s")
    f = functools.partial(
        pl.kernel,
        mesh=mesh,
        compiler_params=pltpu.CompilerParams(
            needs_layout_passes=False, use_tc_tiling_on_sc=False),
        out_type=jax.ShapeDtypeStruct((NCHUNK * _OUT_C,), jnp.float32),
        scratch_types=[
            pltpu.VMEM((V, P, 2, L), jnp.float32),
            pltpu.VMEM((V, P, 2, L), jnp.float32),
            pltpu.VMEM((_SC_C,), jnp.float32),
            pltpu.VMEM((T, 2, L), jnp.float32),
            pltpu.VMEM((_OUT_C,), jnp.float32),
            pltpu.SemaphoreType.DMA,
            pltpu.SemaphoreType.DMA,
        ],
    )(_sc_body)
    return f(ego, lanes5, scores)


def kernel(ego_fut_preds, lane_preds, lane_score_preds):
    lanes5 = (lane_preds.reshape(8, 128, V, P, 2)
              .transpose(2, 3, 0, 4, 1))
    ego = (ego_fut_preds.reshape(8, 128, T, 2)
           .transpose(2, 0, 3, 1))
    scores = (lane_score_preds[:, :, 2].transpose(1, 0)
              .reshape(V, NCHUNK, L).swapaxes(0, 1).reshape(-1))
    out = _sc_call(ego, lanes5, scores).reshape(NCHUNK, 8, L)[:, :T, :]
    d = jnp.sqrt(out)
    loss = jnp.maximum(np.float32(0.0), np.float32(1.0) - d)
    return jnp.mean(loss)
